# baseline (speedup 1.0000x reference)
import functools

import jax
import jax.numpy as jnp
from jax import lax
from jax.experimental import pallas as pl
from jax.experimental.pallas import tpu as pltpu
from jax.experimental.pallas import tpu_sc as plsc

N = 10000
E = 320000
D = 128
NC = 2
NS = 16
NW = NC * NS
EPW = E // NW
K = 80
NCH = EPW // K
G = K // 16
KE = 80
NCH_E = EPW // KE
ROWS_PER_TILE = N // NS


def _rsqrt(x):
    i = plsc.bitcast(x, jnp.int32)
    i = jnp.int32(0x5F3759DF) - lax.shift_right_arithmetic(i, jnp.int32(1))
    y = plsc.bitcast(i, jnp.float32)
    for _ in range(3):
        y = y * (jnp.float32(1.5) - jnp.float32(0.5) * x * y * y)
    return y


def _wid():
    return lax.axis_index("s") * NC + lax.axis_index("c")


_MESH = plsc.VectorSubcoreMesh(
    core_axis_name="c", subcore_axis_name="s", num_cores=NC, num_subcores=NS)
_SC_PARAMS = pltpu.CompilerParams(
    needs_layout_passes=False, use_tc_tiling_on_sc=False)


def _group_dist(px_v, py_v, pz_v, src_v, dst_v, g):
    s16 = src_v[pl.ds(g * 16, 16)]
    d16 = dst_v[pl.ds(g * 16, 16)]
    ex = plsc.load_gather(px_v, [s16]) - plsc.load_gather(px_v, [d16])
    ey = plsc.load_gather(py_v, [s16]) - plsc.load_gather(py_v, [d16])
    ez = plsc.load_gather(pz_v, [s16]) - plsc.load_gather(pz_v, [d16])
    d2 = ex * ex + ey * ey + ez * ez + jnp.float32(1e-8)
    dist = d2 * _rsqrt(d2)
    return ex, ey, ez, dist


def _make_geom_kernel(with_ev):
    if with_ev:
        out_t = [jax.ShapeDtypeStruct((E, 16), jnp.float32),
                 jax.ShapeDtypeStruct((E,), jnp.float32)]
    else:
        out_t = jax.ShapeDtypeStruct((E,), jnp.float32)
    scratch = [
        pltpu.VMEM((N,), jnp.float32),
        pltpu.VMEM((N,), jnp.float32),
        pltpu.VMEM((N,), jnp.float32),
        pltpu.VMEM((K,), jnp.int32),
        pltpu.VMEM((K,), jnp.int32),
        pltpu.VMEM((K,), jnp.float32),
    ]
    if with_ev:
        scratch.insert(5, pltpu.VMEM((K, 16), jnp.float32))

    def body(*refs):
        if with_ev:
            (px_h, py_h, pz_h, src_h, dst_h, ev_h, dist_h,
             px_v, py_v, pz_v, src_v, dst_v, ev_v, d_v) = refs
        else:
            (px_h, py_h, pz_h, src_h, dst_h, dist_h,
             px_v, py_v, pz_v, src_v, dst_v, d_v) = refs
            ev_h = ev_v = None
        wid = _wid()
        pltpu.sync_copy(px_h, px_v)
        pltpu.sync_copy(py_h, py_v)
        pltpu.sync_copy(pz_h, pz_v)
        if with_ev:
            @plsc.parallel_loop(0, K, step=1, unroll=1)
            def zrow(e):
                ev_v[e, pl.ds(0, 16)] = jnp.zeros((16,), jnp.float32)

        def chunk(ch, _):
            base = wid * EPW + ch * K
            pltpu.sync_copy(src_h.at[pl.ds(base, K)], src_v)
            pltpu.sync_copy(dst_h.at[pl.ds(base, K)], dst_v)

            @plsc.parallel_loop(0, G, step=1, unroll=2)
            def grp(g):
                ex, ey, ez, dist = _group_dist(px_v, py_v, pz_v,
                                               src_v, dst_v, g)
                d_v[pl.ds(g * 16, 16)] = dist
                if with_ev:
                    rows = g * 16 + lax.iota(jnp.int32, 16)
                    plsc.store_scatter(
                        ev_v, [rows, jnp.zeros((16,), jnp.int32)], ex)
                    plsc.store_scatter(
                        ev_v, [rows, jnp.full((16,), 1, jnp.int32)], ey)
                    plsc.store_scatter(
                        ev_v, [rows, jnp.full((16,), 2, jnp.int32)], ez)
                    plsc.store_scatter(
                        ev_v, [rows, jnp.full((16,), 3, jnp.int32)], dist)
            if with_ev:
                pltpu.sync_copy(ev_v, ev_h.at[pl.ds(base, K)])
            pltpu.sync_copy(d_v, dist_h.at[pl.ds(base, K)])
            return 0

        lax.fori_loop(0, NCH, chunk, 0)

    return pl.kernel(
        body, out_type=out_t, mesh=_MESH, compiler_params=_SC_PARAMS,
        scratch_types=scratch)


_sc_geom_ev = _make_geom_kernel(True)
_sc_geom_d = _make_geom_kernel(False)


def _edge_buf():
    return dict(
        src=pltpu.VMEM((KE,), jnp.int32),
        dst=pltpu.VMEM((KE,), jnp.int32),
        d=pltpu.VMEM((KE,), jnp.float32),
        a=pltpu.VMEM((KE, D), jnp.float32),
        b=pltpu.VMEM((KE, D), jnp.float32),
        sa=pltpu.SemaphoreType.DMA,
        sb=pltpu.SemaphoreType.DMA,
        so=pltpu.SemaphoreType.DMA,
    )


@functools.partial(
    pl.kernel,
    out_type=jax.ShapeDtypeStruct((NC, N, D), jnp.float32),
    mesh=_MESH,
    compiler_params=_SC_PARAMS,
    scratch_types=[
        _edge_buf(), _edge_buf(),
        pltpu.VMEM((128,), jnp.float32),
        pltpu.VMEM_SHARED((N, D), jnp.float32),
    ],
)
def _sc_edge(a_h, b_h, dist_h, src_h, dst_h, w1r_h, out_h,
             buf0, buf1, w1r_v, seg_s):
    bufs = (buf0, buf1)
    cid = lax.axis_index("c")
    sid = lax.axis_index("s")
    wid = sid * NC + cid
    pltpu.sync_copy(w1r_h, w1r_v)

    a0 = buf0["a"]

    @plsc.parallel_loop(0, 25, step=1, unroll=1)
    def zrow(r):
        for w in range(D // 16):
            a0[r, pl.ds(16 * w, 16)] = jnp.zeros((16,), jnp.float32)

    rbase = sid * ROWS_PER_TILE

    def zcp(z, _):
        pltpu.sync_copy(a0.at[pl.ds(0, 25)],
                        seg_s.at[pl.ds(rbase + z * 25, 25)])
        return 0
    lax.fori_loop(0, ROWS_PER_TILE // 25, zcp, 0)
    plsc.subcore_barrier()

    def start(ch, buf, drain):
        if drain:
            pltpu.make_async_copy(buf["a"], seg_s.at[buf["dst"]],
                                  buf["so"]).wait()
        base = wid * EPW + ch * KE
        pltpu.sync_copy(src_h.at[pl.ds(base, KE)], buf["src"])
        pltpu.sync_copy(dst_h.at[pl.ds(base, KE)], buf["dst"])
        pltpu.async_copy(a_h.at[buf["dst"]], buf["a"], buf["sa"])
        pltpu.async_copy(b_h.at[buf["src"]], buf["b"], buf["sb"])
        pltpu.sync_copy(dist_h.at[pl.ds(base, KE)], buf["d"])

    def process(buf):
        pltpu.make_async_copy(a_h.at[buf["dst"]], buf["a"], buf["sa"]).wait()
        pltpu.make_async_copy(b_h.at[buf["src"]], buf["b"], buf["sb"]).wait()
        a_v, b_v, d_v = buf["a"], buf["b"], buf["d"]

        @plsc.parallel_loop(0, KE, step=1, unroll=8)
        def edge(e):
            dd = plsc.load_gather(d_v, [jnp.full((16,), e, jnp.int32)])
            for j in range(8):
                m = (a_v[e, pl.ds(16 * j, 16)]
                     + b_v[e, pl.ds(16 * j, 16)]
                     + dd * w1r_v[pl.ds(16 * j, 16)])
                a_v[e, pl.ds(16 * j, 16)] = jnp.maximum(m, 0.0)
        pltpu.async_copy(a_v, seg_s.at[buf["dst"]], buf["so"], add=True)

    start(0, bufs[0], False)
    start(1, bufs[1], False)
    process(bufs[0])

    def pair(ii, _):
        start(2 * ii + 2, bufs[0], True)
        process(bufs[1])
        start(2 * ii + 3, bufs[1], True)
        process(bufs[0])
        return 0

    lax.fori_loop(0, (NCH_E - 3) // 2, pair, 0)
    start(NCH_E - 1, bufs[0], True)
    process(bufs[1])
    process(bufs[0])
    pltpu.make_async_copy(bufs[1]["a"], seg_s.at[bufs[1]["dst"]],
                          bufs[1]["so"]).wait()
    pltpu.make_async_copy(bufs[0]["a"], seg_s.at[bufs[0]["dst"]],
                          bufs[0]["so"]).wait()
    plsc.subcore_barrier()
    pltpu.sync_copy(seg_s.at[pl.ds(rbase, ROWS_PER_TILE)],
                    out_h.at[cid, pl.ds(rbase, ROWS_PER_TILE)])


@functools.partial(
    pl.kernel,
    out_type=jax.ShapeDtypeStruct((NC, N, 16), jnp.float32),
    mesh=_MESH,
    compiler_params=_SC_PARAMS,
    scratch_types=[
        pltpu.VMEM((K,), jnp.int32),
        pltpu.VMEM((K, 16), jnp.float32),
        pltpu.VMEM((K,), jnp.float32),
        pltpu.VMEM((K, 16), jnp.float32),
        pltpu.VMEM_SHARED((N, 16), jnp.float32),
    ],
)
def _sc_tail(ev_h, c_h, dst_h, out_h, dst_v, ev_v, c_v, o_v, seg_s):
    cid = lax.axis_index("c")
    sid = lax.axis_index("s")
    wid = sid * NC + cid

    @plsc.parallel_loop(0, K, step=1, unroll=1)
    def zrow(e):
        o_v[e, pl.ds(0, 16)] = jnp.zeros((16,), jnp.float32)

    rbase = sid * ROWS_PER_TILE

    def zcp(z, _):
        pltpu.sync_copy(o_v.at[pl.ds(0, 25)],
                        seg_s.at[pl.ds(rbase + z * 25, 25)])
        return 0
    lax.fori_loop(0, ROWS_PER_TILE // 25, zcp, 0)
    plsc.subcore_barrier()

    lane = lax.iota(jnp.int32, 16)
    mask3 = jnp.where(lane < 3, 1.0, 0.0).astype(jnp.float32)
    unit3 = jnp.where(lane == 3, 1.0, 0.0).astype(jnp.float32)

    def chunk(ch, _):
        base = wid * EPW + ch * K
        pltpu.sync_copy(dst_h.at[pl.ds(base, K)], dst_v)
        pltpu.sync_copy(ev_h.at[pl.ds(base, K)], ev_v)
        pltpu.sync_copy(c_h.at[pl.ds(base, K)], c_v)

        @plsc.parallel_loop(0, K, step=1, unroll=4)
        def edge(e):
            cg = plsc.load_gather(c_v, [jnp.full((16,), e, jnp.int32)])
            evrow = ev_v[e, pl.ds(0, 16)]
            o_v[e, pl.ds(0, 16)] = cg * evrow * mask3 + unit3
        pltpu.sync_copy(o_v, seg_s.at[dst_v], add=True)
        return 0

    lax.fori_loop(0, NCH, chunk, 0)
    plsc.subcore_barrier()
    pltpu.sync_copy(seg_s.at[pl.ds(rbase, ROWS_PER_TILE)],
                    out_h.at[cid, pl.ds(rbase, ROWS_PER_TILE)])


def _bond_buf():
    return dict(
        src=pltpu.VMEM((KE,), jnp.int32),
        dst=pltpu.VMEM((KE,), jnp.int32),
        p=pltpu.VMEM((KE, 64), jnp.float32),
        q=pltpu.VMEM((KE, 64), jnp.float32),
        sa=pltpu.SemaphoreType.DMA,
        sb=pltpu.SemaphoreType.DMA,
        so=pltpu.SemaphoreType.DMA,
    )


@functools.partial(
    pl.kernel,
    out_type=jax.ShapeDtypeStruct((E, 64), jnp.float32),
    mesh=_MESH,
    compiler_params=_SC_PARAMS,
    scratch_types=[_bond_buf(), _bond_buf()],
)
def _sc_bond(p_h, q_h, src_h, dst_h, out_h, buf0, buf1):
    wid = _wid()
    bufs = (buf0, buf1)

    def start(ch, buf, drain, dch):
        if drain:
            dbase = wid * EPW + dch * KE
            pltpu.make_async_copy(buf["p"], out_h.at[pl.ds(dbase, KE)],
                                  buf["so"]).wait()
        base = wid * EPW + ch * KE
        pltpu.sync_copy(src_h.at[pl.ds(base, KE)], buf["src"])
        pltpu.sync_copy(dst_h.at[pl.ds(base, KE)], buf["dst"])
        pltpu.async_copy(p_h.at[buf["src"]], buf["p"], buf["sa"])
        pltpu.async_copy(q_h.at[buf["dst"]], buf["q"], buf["sb"])

    def process(ch, buf):
        pltpu.make_async_copy(p_h.at[buf["src"]], buf["p"], buf["sa"]).wait()
        pltpu.make_async_copy(q_h.at[buf["dst"]], buf["q"], buf["sb"]).wait()
        p_v, q_v = buf["p"], buf["q"]

        @plsc.parallel_loop(0, KE, step=1, unroll=4)
        def edge(e):
            for j in range(4):
                m = p_v[e, pl.ds(16 * j, 16)] + q_v[e, pl.ds(16 * j, 16)]
                p_v[e, pl.ds(16 * j, 16)] = jnp.maximum(m, 0.0)
        base = wid * EPW + ch * KE
        pltpu.async_copy(p_v, out_h.at[pl.ds(base, KE)], buf["so"])

    start(0, bufs[0], False, 0)
    start(1, bufs[1], False, 0)
    process(0, bufs[0])

    def pair(ii, _):
        start(2 * ii + 2, bufs[0], True, 2 * ii)
        process(2 * ii + 1, bufs[1])
        start(2 * ii + 3, bufs[1], True, 2 * ii + 1)
        process(2 * ii + 2, bufs[0])
        return 0

    lax.fori_loop(0, (NCH_E - 3) // 2, pair, 0)
    start(NCH_E - 1, bufs[0], True, NCH_E - 3)
    process(NCH_E - 2, bufs[1])
    process(NCH_E - 1, bufs[0])
    pltpu.make_async_copy(bufs[1]["p"],
                          out_h.at[pl.ds(wid * EPW + (NCH_E - 2) * KE, KE)],
                          bufs[1]["so"]).wait()
    pltpu.make_async_copy(bufs[0]["p"],
                          out_h.at[pl.ds(wid * EPW + (NCH_E - 1) * KE, KE)],
                          bufs[0]["so"]).wait()


_NB = 2000


def _tc_in_body(x_ref, wa_ref, wb_ref, ba_ref, a_ref, b_ref):
    x = x_ref[...]
    a_ref[...] = (jnp.dot(x, wa_ref[...], preferred_element_type=jnp.float32)
                  + ba_ref[...])
    b_ref[...] = jnp.dot(x, wb_ref[...], preferred_element_type=jnp.float32)


def _tc_in(x_in, wa, wb, ba):
    return pl.pallas_call(
        _tc_in_body,
        grid=(N // _NB,),
        in_specs=[
            pl.BlockSpec((_NB, D), lambda i: (i, 0)),
            pl.BlockSpec((D, D), lambda i: (0, 0)),
            pl.BlockSpec((D, D), lambda i: (0, 0)),
            pl.BlockSpec((1, D), lambda i: (0, 0)),
        ],
        out_specs=[
            pl.BlockSpec((_NB, D), lambda i: (i, 0)),
            pl.BlockSpec((_NB, D), lambda i: (i, 0)),
        ],
        out_shape=[
            jax.ShapeDtypeStruct((N, D), jnp.float32),
            jax.ShapeDtypeStruct((N, D), jnp.float32),
        ],
    )(x_in, wa, wb, ba)


def _tc_coord_body(d_ref, w1_ref, b1_ref, w2_ref, b2_ref, c_ref):
    dcol = d_ref[...]
    h = jnp.maximum(
        jnp.dot(dcol, w1_ref[...], preferred_element_type=jnp.float32)
        + b1_ref[...], 0.0)
    c_ref[...] = (jnp.dot(h, w2_ref[...], preferred_element_type=jnp.float32)
                  + b2_ref[...])


def _tc_coord(dist, w1, b1, w2, b2):
    B = 8000
    c = pl.pallas_call(
        _tc_coord_body,
        grid=(E // B,),
        in_specs=[
            pl.BlockSpec((B, 1), lambda i: (i, 0)),
            pl.BlockSpec((1, 128), lambda i: (0, 0)),
            pl.BlockSpec((1, 128), lambda i: (0, 0)),
            pl.BlockSpec((128, 1), lambda i: (0, 0)),
            pl.BlockSpec((1, 1), lambda i: (0, 0)),
        ],
        out_specs=pl.BlockSpec((B, 1), lambda i: (i, 0)),
        out_shape=jax.ShapeDtypeStruct((E, 1), jnp.float32),
    )(dist.reshape(E, 1), w1, b1.reshape(1, 128), w2, b2.reshape(1, 1))
    return c.reshape(E)


def _tc_layer_body(seg_ref, x_ref, a_ref, b_ref, cnt_ref, sb_ref, w2_ref,
                   b2_ref, wa2_ref, wb2_ref, ba2_ref,
                   x1_ref, a2_ref, b2out_ref):
    seg = seg_ref[0] + seg_ref[1]
    selfm = jnp.maximum(a_ref[...] + b_ref[...] + sb_ref[...], 0.0)
    x1 = (x_ref[...]
          + jnp.dot(seg + selfm, w2_ref[...],
                    preferred_element_type=jnp.float32)
          + cnt_ref[...] * b2_ref[...])
    x1_ref[...] = x1
    a2_ref[...] = (jnp.dot(x1, wa2_ref[...],
                           preferred_element_type=jnp.float32) + ba2_ref[...])
    b2out_ref[...] = jnp.dot(x1, wb2_ref[...],
                             preferred_element_type=jnp.float32)


def _tc_layer(seg, x, a, b, cnt, sbias, w2, b2, wa2, wb2, ba2):
    return pl.pallas_call(
        _tc_layer_body,
        grid=(N // _NB,),
        in_specs=[
            pl.BlockSpec((NC, _NB, D), lambda i: (0, i, 0)),
            pl.BlockSpec((_NB, D), lambda i: (i, 0)),
            pl.BlockSpec((_NB, D), lambda i: (i, 0)),
            pl.BlockSpec((_NB, D), lambda i: (i, 0)),
            pl.BlockSpec((_NB, 1), lambda i: (i, 0)),
            pl.BlockSpec((1, D), lambda i: (0, 0)),
            pl.BlockSpec((D, D), lambda i: (0, 0)),
            pl.BlockSpec((1, D), lambda i: (0, 0)),
            pl.BlockSpec((D, D), lambda i: (0, 0)),
            pl.BlockSpec((D, D), lambda i: (0, 0)),
            pl.BlockSpec((1, D), lambda i: (0, 0)),
        ],
        out_specs=[
            pl.BlockSpec((_NB, D), lambda i: (i, 0)),
            pl.BlockSpec((_NB, D), lambda i: (i, 0)),
            pl.BlockSpec((_NB, D), lambda i: (i, 0)),
        ],
        out_shape=[
            jax.ShapeDtypeStruct((N, D), jnp.float32),
            jax.ShapeDtypeStruct((N, D), jnp.float32),
            jax.ShapeDtypeStruct((N, D), jnp.float32),
        ],
    )(seg, x, a, b, cnt, sbias, w2, b2, wa2, wb2, ba2)


def _tc_layer2_body(seg_ref, x_ref, a_ref, b_ref, cnt_ref, sb_ref, w2_ref,
                    b2_ref, wa2_ref, wb2_ref, bp_ref,
                    xc_ref, p_ref, q_ref):
    seg = seg_ref[0] + seg_ref[1]
    selfm = jnp.maximum(a_ref[...] + b_ref[...] + sb_ref[...], 0.0)
    x2 = (x_ref[...]
          + jnp.dot(seg + selfm, w2_ref[...],
                    preferred_element_type=jnp.float32)
          + cnt_ref[...] * b2_ref[...])
    xc_ref[...] = x2[:, :64]
    p_ref[...] = (jnp.dot(x2, wa2_ref[...],
                          preferred_element_type=jnp.float32) + bp_ref[...])
    q_ref[...] = jnp.dot(x2, wb2_ref[...], preferred_element_type=jnp.float32)


def _tc_layer2(seg, x, a, b, cnt, sbias, w2, b2, wa2, wb2, bp):
    return pl.pallas_call(
        _tc_layer2_body,
        grid=(N // _NB,),
        in_specs=[
            pl.BlockSpec((NC, _NB, D), lambda i: (0, i, 0)),
            pl.BlockSpec((_NB, D), lambda i: (i, 0)),
            pl.BlockSpec((_NB, D), lambda i: (i, 0)),
            pl.BlockSpec((_NB, D), lambda i: (i, 0)),
            pl.BlockSpec((_NB, 1), lambda i: (i, 0)),
            pl.BlockSpec((1, D), lambda i: (0, 0)),
            pl.BlockSpec((D, D), lambda i: (0, 0)),
            pl.BlockSpec((1, D), lambda i: (0, 0)),
            pl.BlockSpec((D, 64), lambda i: (0, 0)),
            pl.BlockSpec((D, 64), lambda i: (0, 0)),
            pl.BlockSpec((1, 64), lambda i: (0, 0)),
        ],
        out_specs=[
            pl.BlockSpec((_NB, 64), lambda i: (i, 0)),
            pl.BlockSpec((_NB, 64), lambda i: (i, 0)),
            pl.BlockSpec((_NB, 64), lambda i: (i, 0)),
        ],
        out_shape=[
            jax.ShapeDtypeStruct((N, 64), jnp.float32),
            jax.ShapeDtypeStruct((N, 64), jnp.float32),
            jax.ShapeDtypeStruct((N, 64), jnp.float32),
        ],
    )(seg, x, a, b, cnt, sbias, w2, b2, wa2, wb2, bp)


def _tc_bond_body(h_ref, w2_ref, b2_ref, o_ref):
    o_ref[...] = (jnp.dot(h_ref[...], w2_ref[...],
                          preferred_element_type=jnp.float32) + b2_ref[...])


def _tc_bond(h, w2, b2):
    B = 8000
    return pl.pallas_call(
        _tc_bond_body,
        grid=(E // B,),
        in_specs=[
            pl.BlockSpec((B, 64), lambda i: (i, 0)),
            pl.BlockSpec((64, 4), lambda i: (0, 0)),
            pl.BlockSpec((1, 4), lambda i: (0, 0)),
        ],
        out_specs=pl.BlockSpec((B, 4), lambda i: (i, 0)),
        out_shape=jax.ShapeDtypeStruct((E, 4), jnp.float32),
    )(h, w2, b2.reshape(1, 4))


def kernel(x_t, pos, edge_index, t, cond_embed, params):
    p = params
    n = N
    tt = t.reshape(-1, 1).astype(jnp.float32) / 1000.0
    te = jnp.maximum(tt @ p['te_w1'] + p['te_b1'], 0.0) @ p['te_w2'] + p['te_b2']
    x_in = jnp.concatenate([
        x_t,
        jnp.broadcast_to(cond_embed, (n, cond_embed.shape[1])),
        jnp.broadcast_to(te, (n, te.shape[1])),
    ], axis=1)
    src = edge_index[0]
    dst = edge_index[1]
    d0 = jnp.sqrt(jnp.float32(1e-8))

    px = pos[:, 0]
    py = pos[:, 1]
    pz = pos[:, 2]

    w1r1 = p['e1_nm_w1'][2 * D]
    a1, b1x = _tc_in(x_in, p['e1_nm_w1'][:D], p['e1_nm_w1'][D:2 * D],
                     p['e1_nm_b1'].reshape(1, D))
    ev1, dist1 = _sc_geom_ev(px, py, pz, src, dst)
    c1 = _tc_coord(dist1, p['e1_cm_w1'], p['e1_cm_b1'], p['e1_cm_w2'],
                   p['e1_cm_b2'])
    seg1 = _sc_edge(a1, b1x, dist1, src, dst, w1r1)
    tail1 = _sc_tail(ev1, c1, dst)
    tails = tail1[0, :, :4] + tail1[1, :, :4]
    cnt = (tails[:, 3] + 1.0).reshape(N, 1)
    pos1 = pos + tails[:, :3]
    w1r2 = p['e2_nm_w1'][2 * D]
    x1, a2, b2x = _tc_layer(
        seg1, x_in, a1, b1x, cnt, (d0 * w1r1).reshape(1, D),
        p['e1_nm_w2'], p['e1_nm_b2'].reshape(1, D),
        p['e2_nm_w1'][:D], p['e2_nm_w1'][D:2 * D],
        p['e2_nm_b1'].reshape(1, D))

    dist2 = _sc_geom_d(pos1[:, 0], pos1[:, 1], pos1[:, 2], src, dst)
    seg2 = _sc_edge(a2, b2x, dist2, src, dst, w1r2)
    x2c, pm, qm = _tc_layer2(
        seg2, x1, a2, b2x, cnt, (d0 * w1r2).reshape(1, D),
        p['e2_nm_w2'], p['e2_nm_b2'].reshape(1, D),
        p['bp_w1'][:D], p['bp_w1'][D:2 * D],
        p['bp_b1'].reshape(1, 64))

    h = _sc_bond(pm, qm, src, dst)
    logits = _tc_bond(h, p['bp_w2'], p['bp_b2'])
    return x2c, logits


# speedup vs baseline: 1.0121x; 1.0121x over previous
import functools

import jax
import jax.numpy as jnp
from jax import lax
from jax.experimental import pallas as pl
from jax.experimental.pallas import tpu as pltpu
from jax.experimental.pallas import tpu_sc as plsc

N = 10000
E = 320000
D = 128
NC = 2
NS = 16
NW = NC * NS
EPW = E // NW
K = 80
NCH = EPW // K
G = K // 16
KE = 80
NCH_E = EPW // KE
ROWS_PER_TILE = N // NS


def _rsqrt(x):
    i = plsc.bitcast(x, jnp.int32)
    i = jnp.int32(0x5F3759DF) - lax.shift_right_arithmetic(i, jnp.int32(1))
    y = plsc.bitcast(i, jnp.float32)
    for _ in range(3):
        y = y * (jnp.float32(1.5) - jnp.float32(0.5) * x * y * y)
    return y


def _wid():
    return lax.axis_index("s") * NC + lax.axis_index("c")


_MESH = plsc.VectorSubcoreMesh(
    core_axis_name="c", subcore_axis_name="s", num_cores=NC, num_subcores=NS)
_SC_PARAMS = pltpu.CompilerParams(
    needs_layout_passes=False, use_tc_tiling_on_sc=False)


def _group_dist(px_v, py_v, pz_v, src_v, dst_v, g):
    s16 = src_v[pl.ds(g * 16, 16)]
    d16 = dst_v[pl.ds(g * 16, 16)]
    ex = plsc.load_gather(px_v, [s16]) - plsc.load_gather(px_v, [d16])
    ey = plsc.load_gather(py_v, [s16]) - plsc.load_gather(py_v, [d16])
    ez = plsc.load_gather(pz_v, [s16]) - plsc.load_gather(pz_v, [d16])
    d2 = ex * ex + ey * ey + ez * ez + jnp.float32(1e-8)
    dist = d2 * _rsqrt(d2)
    return ex, ey, ez, dist


def _make_geom_kernel(with_ev):
    if with_ev:
        out_t = [jax.ShapeDtypeStruct((E, 16), jnp.float32),
                 jax.ShapeDtypeStruct((E,), jnp.float32)]
    else:
        out_t = jax.ShapeDtypeStruct((E,), jnp.float32)
    scratch = [
        pltpu.VMEM((N,), jnp.float32),
        pltpu.VMEM((N,), jnp.float32),
        pltpu.VMEM((N,), jnp.float32),
        pltpu.VMEM((K,), jnp.int32),
        pltpu.VMEM((K,), jnp.int32),
        pltpu.VMEM((K,), jnp.float32),
    ]
    if with_ev:
        scratch.insert(5, pltpu.VMEM((K, 16), jnp.float32))

    def body(*refs):
        if with_ev:
            (px_h, py_h, pz_h, src_h, dst_h, ev_h, dist_h,
             px_v, py_v, pz_v, src_v, dst_v, ev_v, d_v) = refs
        else:
            (px_h, py_h, pz_h, src_h, dst_h, dist_h,
             px_v, py_v, pz_v, src_v, dst_v, d_v) = refs
            ev_h = ev_v = None
        wid = _wid()
        pltpu.sync_copy(px_h, px_v)
        pltpu.sync_copy(py_h, py_v)
        pltpu.sync_copy(pz_h, pz_v)
        if with_ev:
            @plsc.parallel_loop(0, K, step=1, unroll=1)
            def zrow(e):
                ev_v[e, pl.ds(0, 16)] = jnp.zeros((16,), jnp.float32)

        def chunk(ch, _):
            base = wid * EPW + ch * K
            pltpu.sync_copy(src_h.at[pl.ds(base, K)], src_v)
            pltpu.sync_copy(dst_h.at[pl.ds(base, K)], dst_v)

            @plsc.parallel_loop(0, G, step=1, unroll=2)
            def grp(g):
                ex, ey, ez, dist = _group_dist(px_v, py_v, pz_v,
                                               src_v, dst_v, g)
                d_v[pl.ds(g * 16, 16)] = dist
                if with_ev:
                    rows = g * 16 + lax.iota(jnp.int32, 16)
                    plsc.store_scatter(
                        ev_v, [rows, jnp.zeros((16,), jnp.int32)], ex)
                    plsc.store_scatter(
                        ev_v, [rows, jnp.full((16,), 1, jnp.int32)], ey)
                    plsc.store_scatter(
                        ev_v, [rows, jnp.full((16,), 2, jnp.int32)], ez)
                    plsc.store_scatter(
                        ev_v, [rows, jnp.full((16,), 3, jnp.int32)], dist)
            if with_ev:
                pltpu.sync_copy(ev_v, ev_h.at[pl.ds(base, K)])
            pltpu.sync_copy(d_v, dist_h.at[pl.ds(base, K)])
            return 0

        lax.fori_loop(0, NCH, chunk, 0)

    return pl.kernel(
        body, out_type=out_t, mesh=_MESH, compiler_params=_SC_PARAMS,
        scratch_types=scratch)


_sc_geom_ev = _make_geom_kernel(True)
_sc_geom_d = _make_geom_kernel(False)


def _edge_buf():
    return dict(
        src=pltpu.VMEM((KE,), jnp.int32),
        dst=pltpu.VMEM((KE,), jnp.int32),
        d=pltpu.VMEM((KE,), jnp.float32),
        a=pltpu.VMEM((KE, D), jnp.float32),
        b=pltpu.VMEM((KE, D), jnp.float32),
        sa=pltpu.SemaphoreType.DMA,
        sb=pltpu.SemaphoreType.DMA,
        so=pltpu.SemaphoreType.DMA,
    )


@functools.partial(
    pl.kernel,
    out_type=jax.ShapeDtypeStruct((NC, N, D), jnp.float32),
    mesh=_MESH,
    compiler_params=_SC_PARAMS,
    scratch_types=[
        _edge_buf(), _edge_buf(),
        pltpu.VMEM((128,), jnp.float32),
        pltpu.VMEM_SHARED((N, D), jnp.float32),
    ],
)
def _sc_edge(a_h, b_h, dist_h, src_h, dst_h, w1r_h, out_h,
             buf0, buf1, w1r_v, seg_s):
    bufs = (buf0, buf1)
    cid = lax.axis_index("c")
    sid = lax.axis_index("s")
    wid = sid * NC + cid
    pltpu.sync_copy(w1r_h, w1r_v)

    a0 = buf0["a"]

    @plsc.parallel_loop(0, 25, step=1, unroll=1)
    def zrow(r):
        for w in range(D // 16):
            a0[r, pl.ds(16 * w, 16)] = jnp.zeros((16,), jnp.float32)

    rbase = sid * ROWS_PER_TILE

    def zcp(z, _):
        pltpu.sync_copy(a0.at[pl.ds(0, 25)],
                        seg_s.at[pl.ds(rbase + z * 25, 25)])
        return 0
    lax.fori_loop(0, ROWS_PER_TILE // 25, zcp, 0)
    plsc.subcore_barrier()

    def start(ch, buf, drain):
        if drain:
            pltpu.make_async_copy(buf["a"], seg_s.at[buf["dst"]],
                                  buf["so"]).wait()
        base = wid * EPW + ch * KE
        pltpu.sync_copy(src_h.at[pl.ds(base, KE)], buf["src"])
        pltpu.sync_copy(dst_h.at[pl.ds(base, KE)], buf["dst"])
        pltpu.async_copy(a_h.at[buf["dst"]], buf["a"], buf["sa"])
        pltpu.async_copy(b_h.at[buf["src"]], buf["b"], buf["sb"])
        pltpu.sync_copy(dist_h.at[pl.ds(base, KE)], buf["d"])

    def process(buf):
        pltpu.make_async_copy(a_h.at[buf["dst"]], buf["a"], buf["sa"]).wait()
        pltpu.make_async_copy(b_h.at[buf["src"]], buf["b"], buf["sb"]).wait()
        a_v, b_v, d_v = buf["a"], buf["b"], buf["d"]

        @plsc.parallel_loop(0, KE, step=1, unroll=4)
        def edge(e):
            dd = plsc.load_gather(d_v, [jnp.full((16,), e, jnp.int32)])
            for j in range(8):
                m = (a_v[e, pl.ds(16 * j, 16)]
                     + b_v[e, pl.ds(16 * j, 16)]
                     + dd * w1r_v[pl.ds(16 * j, 16)])
                a_v[e, pl.ds(16 * j, 16)] = jnp.maximum(m, 0.0)
        pltpu.async_copy(a_v, seg_s.at[buf["dst"]], buf["so"], add=True)

    start(0, bufs[0], False)
    start(1, bufs[1], False)
    process(bufs[0])

    def pair(ii, _):
        start(2 * ii + 2, bufs[0], True)
        process(bufs[1])
        start(2 * ii + 3, bufs[1], True)
        process(bufs[0])
        return 0

    lax.fori_loop(0, (NCH_E - 3) // 2, pair, 0)
    start(NCH_E - 1, bufs[0], True)
    process(bufs[1])
    process(bufs[0])
    pltpu.make_async_copy(bufs[1]["a"], seg_s.at[bufs[1]["dst"]],
                          bufs[1]["so"]).wait()
    pltpu.make_async_copy(bufs[0]["a"], seg_s.at[bufs[0]["dst"]],
                          bufs[0]["so"]).wait()
    plsc.subcore_barrier()
    pltpu.sync_copy(seg_s.at[pl.ds(rbase, ROWS_PER_TILE)],
                    out_h.at[cid, pl.ds(rbase, ROWS_PER_TILE)])


@functools.partial(
    pl.kernel,
    out_type=jax.ShapeDtypeStruct((NC, N, 16), jnp.float32),
    mesh=_MESH,
    compiler_params=_SC_PARAMS,
    scratch_types=[
        pltpu.VMEM((K,), jnp.int32),
        pltpu.VMEM((K, 16), jnp.float32),
        pltpu.VMEM((K,), jnp.float32),
        pltpu.VMEM((K, 16), jnp.float32),
        pltpu.VMEM_SHARED((N, 16), jnp.float32),
    ],
)
def _sc_tail(ev_h, c_h, dst_h, out_h, dst_v, ev_v, c_v, o_v, seg_s):
    cid = lax.axis_index("c")
    sid = lax.axis_index("s")
    wid = sid * NC + cid

    @plsc.parallel_loop(0, K, step=1, unroll=1)
    def zrow(e):
        o_v[e, pl.ds(0, 16)] = jnp.zeros((16,), jnp.float32)

    rbase = sid * ROWS_PER_TILE

    def zcp(z, _):
        pltpu.sync_copy(o_v.at[pl.ds(0, 25)],
                        seg_s.at[pl.ds(rbase + z * 25, 25)])
        return 0
    lax.fori_loop(0, ROWS_PER_TILE // 25, zcp, 0)
    plsc.subcore_barrier()

    lane = lax.iota(jnp.int32, 16)
    mask3 = jnp.where(lane < 3, 1.0, 0.0).astype(jnp.float32)
    unit3 = jnp.where(lane == 3, 1.0, 0.0).astype(jnp.float32)

    def chunk(ch, _):
        base = wid * EPW + ch * K
        pltpu.sync_copy(dst_h.at[pl.ds(base, K)], dst_v)
        pltpu.sync_copy(ev_h.at[pl.ds(base, K)], ev_v)
        pltpu.sync_copy(c_h.at[pl.ds(base, K)], c_v)

        @plsc.parallel_loop(0, K, step=1, unroll=4)
        def edge(e):
            cg = plsc.load_gather(c_v, [jnp.full((16,), e, jnp.int32)])
            evrow = ev_v[e, pl.ds(0, 16)]
            o_v[e, pl.ds(0, 16)] = cg * evrow * mask3 + unit3
        pltpu.sync_copy(o_v, seg_s.at[dst_v], add=True)
        return 0

    lax.fori_loop(0, NCH, chunk, 0)
    plsc.subcore_barrier()
    pltpu.sync_copy(seg_s.at[pl.ds(rbase, ROWS_PER_TILE)],
                    out_h.at[cid, pl.ds(rbase, ROWS_PER_TILE)])


def _bond_buf():
    return dict(
        src=pltpu.VMEM((KE,), jnp.int32),
        dst=pltpu.VMEM((KE,), jnp.int32),
        p=pltpu.VMEM((KE, 64), jnp.float32),
        q=pltpu.VMEM((KE, 64), jnp.float32),
        sa=pltpu.SemaphoreType.DMA,
        sb=pltpu.SemaphoreType.DMA,
        so=pltpu.SemaphoreType.DMA,
    )


@functools.partial(
    pl.kernel,
    out_type=jax.ShapeDtypeStruct((E, 64), jnp.float32),
    mesh=_MESH,
    compiler_params=_SC_PARAMS,
    scratch_types=[_bond_buf(), _bond_buf()],
)
def _sc_bond(p_h, q_h, src_h, dst_h, out_h, buf0, buf1):
    wid = _wid()
    bufs = (buf0, buf1)

    def start(ch, buf, drain, dch):
        if drain:
            dbase = wid * EPW + dch * KE
            pltpu.make_async_copy(buf["p"], out_h.at[pl.ds(dbase, KE)],
                                  buf["so"]).wait()
        base = wid * EPW + ch * KE
        pltpu.sync_copy(src_h.at[pl.ds(base, KE)], buf["src"])
        pltpu.sync_copy(dst_h.at[pl.ds(base, KE)], buf["dst"])
        pltpu.async_copy(p_h.at[buf["src"]], buf["p"], buf["sa"])
        pltpu.async_copy(q_h.at[buf["dst"]], buf["q"], buf["sb"])

    def process(ch, buf):
        pltpu.make_async_copy(p_h.at[buf["src"]], buf["p"], buf["sa"]).wait()
        pltpu.make_async_copy(q_h.at[buf["dst"]], buf["q"], buf["sb"]).wait()
        p_v, q_v = buf["p"], buf["q"]

        @plsc.parallel_loop(0, KE, step=1, unroll=4)
        def edge(e):
            for j in range(4):
                m = p_v[e, pl.ds(16 * j, 16)] + q_v[e, pl.ds(16 * j, 16)]
                p_v[e, pl.ds(16 * j, 16)] = jnp.maximum(m, 0.0)
        base = wid * EPW + ch * KE
        pltpu.async_copy(p_v, out_h.at[pl.ds(base, KE)], buf["so"])

    start(0, bufs[0], False, 0)
    start(1, bufs[1], False, 0)
    process(0, bufs[0])

    def pair(ii, _):
        start(2 * ii + 2, bufs[0], True, 2 * ii)
        process(2 * ii + 1, bufs[1])
        start(2 * ii + 3, bufs[1], True, 2 * ii + 1)
        process(2 * ii + 2, bufs[0])
        return 0

    lax.fori_loop(0, (NCH_E - 3) // 2, pair, 0)
    start(NCH_E - 1, bufs[0], True, NCH_E - 3)
    process(NCH_E - 2, bufs[1])
    process(NCH_E - 1, bufs[0])
    pltpu.make_async_copy(bufs[1]["p"],
                          out_h.at[pl.ds(wid * EPW + (NCH_E - 2) * KE, KE)],
                          bufs[1]["so"]).wait()
    pltpu.make_async_copy(bufs[0]["p"],
                          out_h.at[pl.ds(wid * EPW + (NCH_E - 1) * KE, KE)],
                          bufs[0]["so"]).wait()


_NB = 2000


def _tc_in_body(x_ref, wa_ref, wb_ref, ba_ref, a_ref, b_ref):
    x = x_ref[...]
    a_ref[...] = (jnp.dot(x, wa_ref[...], preferred_element_type=jnp.float32)
                  + ba_ref[...])
    b_ref[...] = jnp.dot(x, wb_ref[...], preferred_element_type=jnp.float32)


def _tc_in(x_in, wa, wb, ba):
    return pl.pallas_call(
        _tc_in_body,
        grid=(N // _NB,),
        in_specs=[
            pl.BlockSpec((_NB, D), lambda i: (i, 0)),
            pl.BlockSpec((D, D), lambda i: (0, 0)),
            pl.BlockSpec((D, D), lambda i: (0, 0)),
            pl.BlockSpec((1, D), lambda i: (0, 0)),
        ],
        out_specs=[
            pl.BlockSpec((_NB, D), lambda i: (i, 0)),
            pl.BlockSpec((_NB, D), lambda i: (i, 0)),
        ],
        out_shape=[
            jax.ShapeDtypeStruct((N, D), jnp.float32),
            jax.ShapeDtypeStruct((N, D), jnp.float32),
        ],
    )(x_in, wa, wb, ba)


def _tc_coord_body(d_ref, w1_ref, b1_ref, w2_ref, b2_ref, c_ref):
    dcol = d_ref[...]
    h = jnp.maximum(
        jnp.dot(dcol, w1_ref[...], preferred_element_type=jnp.float32)
        + b1_ref[...], 0.0)
    c_ref[...] = (jnp.dot(h, w2_ref[...], preferred_element_type=jnp.float32)
                  + b2_ref[...])


def _tc_coord(dist, w1, b1, w2, b2):
    B = 8000
    c = pl.pallas_call(
        _tc_coord_body,
        grid=(E // B,),
        in_specs=[
            pl.BlockSpec((B, 1), lambda i: (i, 0)),
            pl.BlockSpec((1, 128), lambda i: (0, 0)),
            pl.BlockSpec((1, 128), lambda i: (0, 0)),
            pl.BlockSpec((128, 1), lambda i: (0, 0)),
            pl.BlockSpec((1, 1), lambda i: (0, 0)),
        ],
        out_specs=pl.BlockSpec((B, 1), lambda i: (i, 0)),
        out_shape=jax.ShapeDtypeStruct((E, 1), jnp.float32),
    )(dist.reshape(E, 1), w1, b1.reshape(1, 128), w2, b2.reshape(1, 1))
    return c.reshape(E)


def _tc_layer_body(seg_ref, x_ref, a_ref, b_ref, cnt_ref, sb_ref, w2_ref,
                   b2_ref, wa2_ref, wb2_ref, ba2_ref,
                   x1_ref, a2_ref, b2out_ref):
    seg = seg_ref[0] + seg_ref[1]
    selfm = jnp.maximum(a_ref[...] + b_ref[...] + sb_ref[...], 0.0)
    x1 = (x_ref[...]
          + jnp.dot(seg + selfm, w2_ref[...],
                    preferred_element_type=jnp.float32)
          + cnt_ref[...] * b2_ref[...])
    x1_ref[...] = x1
    a2_ref[...] = (jnp.dot(x1, wa2_ref[...],
                           preferred_element_type=jnp.float32) + ba2_ref[...])
    b2out_ref[...] = jnp.dot(x1, wb2_ref[...],
                             preferred_element_type=jnp.float32)


def _tc_layer(seg, x, a, b, cnt, sbias, w2, b2, wa2, wb2, ba2):
    return pl.pallas_call(
        _tc_layer_body,
        grid=(N // _NB,),
        in_specs=[
            pl.BlockSpec((NC, _NB, D), lambda i: (0, i, 0)),
            pl.BlockSpec((_NB, D), lambda i: (i, 0)),
            pl.BlockSpec((_NB, D), lambda i: (i, 0)),
            pl.BlockSpec((_NB, D), lambda i: (i, 0)),
            pl.BlockSpec((_NB, 1), lambda i: (i, 0)),
            pl.BlockSpec((1, D), lambda i: (0, 0)),
            pl.BlockSpec((D, D), lambda i: (0, 0)),
            pl.BlockSpec((1, D), lambda i: (0, 0)),
            pl.BlockSpec((D, D), lambda i: (0, 0)),
            pl.BlockSpec((D, D), lambda i: (0, 0)),
            pl.BlockSpec((1, D), lambda i: (0, 0)),
        ],
        out_specs=[
            pl.BlockSpec((_NB, D), lambda i: (i, 0)),
            pl.BlockSpec((_NB, D), lambda i: (i, 0)),
            pl.BlockSpec((_NB, D), lambda i: (i, 0)),
        ],
        out_shape=[
            jax.ShapeDtypeStruct((N, D), jnp.float32),
            jax.ShapeDtypeStruct((N, D), jnp.float32),
            jax.ShapeDtypeStruct((N, D), jnp.float32),
        ],
    )(seg, x, a, b, cnt, sbias, w2, b2, wa2, wb2, ba2)


def _tc_layer2_body(seg_ref, x_ref, a_ref, b_ref, cnt_ref, sb_ref, w2_ref,
                    b2_ref, wa2_ref, wb2_ref, bp_ref,
                    xc_ref, p_ref, q_ref):
    seg = seg_ref[0] + seg_ref[1]
    selfm = jnp.maximum(a_ref[...] + b_ref[...] + sb_ref[...], 0.0)
    x2 = (x_ref[...]
          + jnp.dot(seg + selfm, w2_ref[...],
                    preferred_element_type=jnp.float32)
          + cnt_ref[...] * b2_ref[...])
    xc_ref[...] = x2[:, :64]
    p_ref[...] = (jnp.dot(x2, wa2_ref[...],
                          preferred_element_type=jnp.float32) + bp_ref[...])
    q_ref[...] = jnp.dot(x2, wb2_ref[...], preferred_element_type=jnp.float32)


def _tc_layer2(seg, x, a, b, cnt, sbias, w2, b2, wa2, wb2, bp):
    return pl.pallas_call(
        _tc_layer2_body,
        grid=(N // _NB,),
        in_specs=[
            pl.BlockSpec((NC, _NB, D), lambda i: (0, i, 0)),
            pl.BlockSpec((_NB, D), lambda i: (i, 0)),
            pl.BlockSpec((_NB, D), lambda i: (i, 0)),
            pl.BlockSpec((_NB, D), lambda i: (i, 0)),
            pl.BlockSpec((_NB, 1), lambda i: (i, 0)),
            pl.BlockSpec((1, D), lambda i: (0, 0)),
            pl.BlockSpec((D, D), lambda i: (0, 0)),
            pl.BlockSpec((1, D), lambda i: (0, 0)),
            pl.BlockSpec((D, 64), lambda i: (0, 0)),
            pl.BlockSpec((D, 64), lambda i: (0, 0)),
            pl.BlockSpec((1, 64), lambda i: (0, 0)),
        ],
        out_specs=[
            pl.BlockSpec((_NB, 64), lambda i: (i, 0)),
            pl.BlockSpec((_NB, 64), lambda i: (i, 0)),
            pl.BlockSpec((_NB, 64), lambda i: (i, 0)),
        ],
        out_shape=[
            jax.ShapeDtypeStruct((N, 64), jnp.float32),
            jax.ShapeDtypeStruct((N, 64), jnp.float32),
            jax.ShapeDtypeStruct((N, 64), jnp.float32),
        ],
    )(seg, x, a, b, cnt, sbias, w2, b2, wa2, wb2, bp)


def _tc_bond_body(h_ref, w2_ref, b2_ref, o_ref):
    o_ref[...] = (jnp.dot(h_ref[...], w2_ref[...],
                          preferred_element_type=jnp.float32) + b2_ref[...])


def _tc_bond(h, w2, b2):
    B = 8000
    return pl.pallas_call(
        _tc_bond_body,
        grid=(E // B,),
        in_specs=[
            pl.BlockSpec((B, 64), lambda i: (i, 0)),
            pl.BlockSpec((64, 4), lambda i: (0, 0)),
            pl.BlockSpec((1, 4), lambda i: (0, 0)),
        ],
        out_specs=pl.BlockSpec((B, 4), lambda i: (i, 0)),
        out_shape=jax.ShapeDtypeStruct((E, 4), jnp.float32),
    )(h, w2, b2.reshape(1, 4))


def kernel(x_t, pos, edge_index, t, cond_embed, params):
    p = params
    n = N
    tt = t.reshape(-1, 1).astype(jnp.float32) / 1000.0
    te = jnp.maximum(tt @ p['te_w1'] + p['te_b1'], 0.0) @ p['te_w2'] + p['te_b2']
    x_in = jnp.concatenate([
        x_t,
        jnp.broadcast_to(cond_embed, (n, cond_embed.shape[1])),
        jnp.broadcast_to(te, (n, te.shape[1])),
    ], axis=1)
    src = edge_index[0]
    dst = edge_index[1]
    d0 = jnp.sqrt(jnp.float32(1e-8))

    px = pos[:, 0]
    py = pos[:, 1]
    pz = pos[:, 2]

    w1r1 = p['e1_nm_w1'][2 * D]
    a1, b1x = _tc_in(x_in, p['e1_nm_w1'][:D], p['e1_nm_w1'][D:2 * D],
                     p['e1_nm_b1'].reshape(1, D))
    ev1, dist1 = _sc_geom_ev(px, py, pz, src, dst)
    c1 = _tc_coord(dist1, p['e1_cm_w1'], p['e1_cm_b1'], p['e1_cm_w2'],
                   p['e1_cm_b2'])
    seg1 = _sc_edge(a1, b1x, dist1, src, dst, w1r1)
    tail1 = _sc_tail(ev1, c1, dst)
    tails = tail1[0, :, :4] + tail1[1, :, :4]
    cnt = (tails[:, 3] + 1.0).reshape(N, 1)
    pos1 = pos + tails[:, :3]
    w1r2 = p['e2_nm_w1'][2 * D]
    x1, a2, b2x = _tc_layer(
        seg1, x_in, a1, b1x, cnt, (d0 * w1r1).reshape(1, D),
        p['e1_nm_w2'], p['e1_nm_b2'].reshape(1, D),
        p['e2_nm_w1'][:D], p['e2_nm_w1'][D:2 * D],
        p['e2_nm_b1'].reshape(1, D))

    dist2 = _sc_geom_d(pos1[:, 0], pos1[:, 1], pos1[:, 2], src, dst)
    seg2 = _sc_edge(a2, b2x, dist2, src, dst, w1r2)
    x2c, pm, qm = _tc_layer2(
        seg2, x1, a2, b2x, cnt, (d0 * w1r2).reshape(1, D),
        p['e2_nm_w2'], p['e2_nm_b2'].reshape(1, D),
        p['bp_w1'][:D], p['bp_w1'][D:2 * D],
        p['bp_b1'].reshape(1, 64))

    h = _sc_bond(pm, qm, src, dst)
    logits = _tc_bond(h, p['bp_w2'], p['bp_b2'])
    return x2c, logits
